# baseline (speedup 1.0000x reference)
import functools

import jax
import jax.numpy as jnp
from jax.experimental import pallas as pl
from jax.experimental.pallas import tpu as pltpu
from jax.experimental.pallas import tpu_sc as plsc


def _sc_embed(exp1, table):
    rows, cols = table.shape
    mesh = plsc.ScalarSubcoreMesh(axis_name="c", num_cores=2)

    @functools.partial(
        pl.kernel,
        mesh=mesh,
        out_type=jax.ShapeDtypeStruct((cols,), table.dtype),
        scratch_types=[
            pltpu.SMEM((1,), jnp.int32),
            pltpu.SemaphoreType.DMA,
        ],
        compiler_params=pltpu.CompilerParams(
            skip_device_barrier=True,
        ),
    )
    def k(exp_hbm, table_hbm, out_hbm, exp_s, sem):
        @pl.when(jax.lax.axis_index("c") == 0)
        def _():
            pltpu.async_copy(exp_hbm, exp_s, sem).wait()
            row = jnp.clip(exp_s[0], -20, 20) + 20
            pltpu.async_copy(table_hbm.at[row], out_hbm, sem).wait()

    return k(exp1, table)


def kernel(exponent, E):
    exp1 = jnp.asarray(exponent, jnp.int32).reshape(1)
    return _sc_embed(exp1, E)


# speedup vs baseline: 1.0558x; 1.0558x over previous
import functools

import jax
import jax.numpy as jnp
from jax.experimental import pallas as pl
from jax.experimental.pallas import tpu as pltpu
from jax.experimental.pallas import tpu_sc as plsc


def _sc_embed(exp1, table):
    rows, cols = table.shape
    mesh = plsc.ScalarSubcoreMesh(axis_name="c", num_cores=1)

    @functools.partial(
        pl.kernel,
        mesh=mesh,
        out_type=jax.ShapeDtypeStruct((cols,), table.dtype),
        scratch_types=[
            pltpu.SMEM((1,), jnp.int32),
            pltpu.SemaphoreType.DMA,
        ],
    )
    def k(exp_hbm, table_hbm, out_hbm, exp_s, sem):
        pltpu.async_copy(exp_hbm, exp_s, sem).wait()
        row = jnp.clip(exp_s[0], -20, 20) + 20
        pltpu.async_copy(table_hbm.at[row], out_hbm, sem).wait()

    return k(exp1, table)


def kernel(exponent, E):
    exp1 = jnp.asarray(exponent, jnp.int32).reshape(1)
    return _sc_embed(exp1, E)
